# baseline (speedup 1.0000x reference)
import functools

import jax
import jax.numpy as jnp
from jax import lax
from jax.experimental import pallas as pl
from jax.experimental.pallas import tpu as pltpu
from jax.experimental.pallas import tpu_sc as plsc

_B = 4
_N = 4096
_D = 32
_K = 16
_WEIGHT = 1.0
_TOTAL = _B * _N

_R = 512
_INF = float("inf")


def _knn_body(cq_ref, ca_ref, idx_ref, w_ref):
    b = pl.program_id(0)
    i = pl.program_id(1)

    cq = cq_ref[0]
    ca = ca_ref[0]

    dx = cq[:, 0:1] - ca[0:1, :]
    dy = cq[:, 1:2] - ca[1:2, :]
    dz = cq[:, 2:3] - ca[2:3, :]
    d2 = dx * dx + dy * dy + dz * dz

    cols = lax.broadcasted_iota(jnp.int32, (_R, _N), 1)
    d2i = lax.bitcast_convert_type(d2, jnp.int32)
    keyi = (d2i & jnp.int32(-4096)) | cols
    sent = jnp.float32(3.4028235e38)

    row_col = i * _R + lax.broadcasted_iota(jnp.int32, (_R, _N), 0)
    key = jnp.where(cols == row_col, sent,
                    lax.bitcast_convert_type(keyi, jnp.float32))

    q = _N // 4

    def _ce(a, b):
        return jnp.minimum(a, b), jnp.maximum(a, b)

    s0, s1 = _ce(key[:, 0:q], key[:, q:2 * q])
    s2, s3 = _ce(key[:, 2 * q:3 * q], key[:, 3 * q:4 * q])
    s0, s2 = _ce(s0, s2)
    s1, s3 = _ce(s1, s3)
    s1, s2 = _ce(s1, s2)

    keys = []
    for _ in range(_K - 1):
        m = jnp.min(s0, axis=1, keepdims=True)
        keys.append(m)
        eq = s0 == m
        s0 = jnp.where(eq, s1, s0)
        s1 = jnp.where(eq, s2, s1)
        s2 = jnp.where(eq, s3, s2)
        s3 = jnp.where(eq, sent, s3)

    packed = lax.bitcast_convert_type(
        jnp.concatenate(keys, axis=1), jnp.int32)
    nbr = packed & jnp.int32(0xFFF)
    v = lax.bitcast_convert_type(packed & jnp.int32(-4096), jnp.float32)
    w = 1.0 / (jnp.sqrt(v) + 1e-5)

    row_local = i * _R + lax.broadcasted_iota(jnp.int32, (_R, 1), 0)
    idx_ref[0] = jnp.concatenate([row_local, nbr], axis=1) + b * _N
    w_ref[0] = jnp.concatenate([jnp.zeros((_R, 1), jnp.float32), w], axis=1)


def _knn(coords):
    coords_t = jnp.transpose(coords, (0, 2, 1))
    grid = (_B, _N // _R)
    return pl.pallas_call(
        _knn_body,
        grid=grid,
        in_specs=[
            pl.BlockSpec((1, _R, 3), lambda b, i: (b, i, 0)),
            pl.BlockSpec((1, 3, _N), lambda b, i: (b, 0, 0)),
        ],
        out_specs=[
            pl.BlockSpec((1, _R, _K), lambda b, i: (b, i, 0)),
            pl.BlockSpec((1, _R, _K), lambda b, i: (b, i, 0)),
        ],
        out_shape=[
            jax.ShapeDtypeStruct((_B, _N, _K), jnp.int32),
            jax.ShapeDtypeStruct((_B, _N, _K), jnp.float32),
        ],
    )(coords, coords_t)


_NC = 2
_NS = 16
_NW = _NC * _NS
_PW = _TOTAL // _NW
_CH = 64
_GL = 128
_NG = (_CH * _K) // _GL


def _tv_body(feats_hbm, idx_hbm, w_hbm, tv_hbm, idx_v, rows_v, of_v, w_v,
             tv_v, sem):
    wid = lax.axis_index("s") * _NC + lax.axis_index("c")

    def chunk_body(c, carry):
        base_pt = pl.multiple_of(wid * _PW + c * _CH, _CH)
        base_row = pl.multiple_of(base_pt * _K // _GL, 8)
        pltpu.sync_copy(idx_hbm.at[pl.ds(base_row, _NG)], idx_v)
        pltpu.sync_copy(w_hbm.at[pl.ds(base_pt * _K, _CH * _K)], w_v)
        pltpu.sync_copy(feats_hbm.at[pl.ds(base_pt, _CH)], of_v)
        copies = []
        for g in range(_NG):
            copies.append(
                pltpu.async_copy(
                    feats_hbm.at[idx_v.at[g]],
                    rows_v.at[pl.ds(g * _GL, _GL)],
                    sem,
                ))
        for cp in copies:
            cp.wait()

        lane = lax.broadcasted_iota(jnp.int32, (16,), 0)

        def pt_body(p, vec):
            o0 = of_v[p, pl.ds(0, 16)]
            o1 = of_v[p, pl.ds(16, 16)]
            wk = w_v[pl.ds(p * _K, _K)]
            acc = jnp.zeros((16,), jnp.float32)
            for k in range(_K):
                r = p * _K + k
                d = (jnp.abs(rows_v[r, pl.ds(0, 16)] - o0)
                     + jnp.abs(rows_v[r, pl.ds(16, 16)] - o1))
                acc = acc + d * wk[k]
            for sh in (8, 4, 2, 1):
                acc = acc + acc.at[lane ^ sh].get(mode="promise_in_bounds")
            vec = jnp.where(lane == (p & 15), acc, vec)

            @pl.when((p & 15) == 15)
            def _store():
                tv_v[pl.ds(p - 15, 16)] = vec

            return vec

        lax.fori_loop(0, _CH, pt_body, jnp.zeros((16,), jnp.float32))
        pltpu.sync_copy(tv_v, tv_hbm.at[pl.ds(base_pt, _CH)])
        return carry

    lax.fori_loop(0, _PW // _CH, chunk_body, 0)


@functools.partial(jax.jit, static_argnames=())
def _tv_sc(feats_flat, idx_flat2d, w_flat):
    kern = pl.kernel(
        _tv_body,
        out_type=jax.ShapeDtypeStruct((_TOTAL,), jnp.float32),
        mesh=plsc.VectorSubcoreMesh(core_axis_name="c", subcore_axis_name="s"),
        scratch_types=[
            pltpu.VMEM((_NG, _GL), jnp.int32),
            pltpu.VMEM((_CH * _K, _D), jnp.float32),
            pltpu.VMEM((_CH, _D), jnp.float32),
            pltpu.VMEM((_CH * _K,), jnp.float32),
            pltpu.VMEM((_CH,), jnp.float32),
            pltpu.SemaphoreType.DMA,
        ],
        compiler_params=pltpu.CompilerParams(use_tc_tiling_on_sc=False),
    )
    return kern(feats_flat, idx_flat2d, w_flat)


def kernel(coords, feats, iteration=0):
    coords = lax.stop_gradient(coords)
    nbr_idx, w = _knn(coords)

    feats_flat = feats.reshape(_TOTAL, _D)
    idx_flat2d = nbr_idx.reshape(_TOTAL * _K // _GL, _GL)
    w_flat = w.reshape(_TOTAL * _K)

    tv_flat = _tv_sc(feats_flat, idx_flat2d, w_flat)
    tv = tv_flat.reshape(_B, _N) * _WEIGHT
    total_loss = tv.mean() + jnp.asarray(iteration, jnp.float32) * 0.0
    return (total_loss, tv)


# speedup vs baseline: 1.0189x; 1.0189x over previous
import functools

import jax
import jax.numpy as jnp
from jax import lax
from jax.experimental import pallas as pl
from jax.experimental.pallas import tpu as pltpu
from jax.experimental.pallas import tpu_sc as plsc

_B = 4
_N = 4096
_D = 32
_K = 16
_WEIGHT = 1.0
_TOTAL = _B * _N

_R = 512
_INF = float("inf")


def _knn_body(cq_ref, ca_ref, idx_ref, w_ref):
    b = pl.program_id(0)
    i = pl.program_id(1)

    cq = cq_ref[0]
    ca = ca_ref[0]

    dx = cq[:, 0:1] - ca[0:1, :]
    dy = cq[:, 1:2] - ca[1:2, :]
    dz = cq[:, 2:3] - ca[2:3, :]
    d2 = dx * dx + dy * dy + dz * dz

    cols = lax.broadcasted_iota(jnp.int32, (_R, _N), 1)
    d2i = lax.bitcast_convert_type(d2, jnp.int32)
    keyi = (d2i & jnp.int32(-4096)) | cols
    sent = jnp.float32(3.4028235e38)

    row_col = i * _R + lax.broadcasted_iota(jnp.int32, (_R, _N), 0)
    key = jnp.where(cols == row_col, sent,
                    lax.bitcast_convert_type(keyi, jnp.float32))

    h = _N // 2
    lo = jnp.minimum(key[:, 0:h], key[:, h:_N])
    hi = jnp.maximum(key[:, 0:h], key[:, h:_N])

    keys = []
    for _ in range(_K - 1):
        m = jnp.min(lo, axis=1, keepdims=True)
        keys.append(m)
        eq = lo == m
        lo = jnp.where(eq, hi, lo)
        hi = jnp.where(eq, sent, hi)

    packed = lax.bitcast_convert_type(
        jnp.concatenate(keys, axis=1), jnp.int32)
    nbr = packed & jnp.int32(0xFFF)
    v = lax.bitcast_convert_type(packed & jnp.int32(-4096), jnp.float32)
    w = 1.0 / (jnp.sqrt(v) + 1e-5)

    row_local = i * _R + lax.broadcasted_iota(jnp.int32, (_R, 1), 0)
    idx_ref[0] = jnp.concatenate([row_local, nbr], axis=1) + b * _N
    w_ref[0] = jnp.concatenate([jnp.zeros((_R, 1), jnp.float32), w], axis=1)


def _knn(coords):
    coords_t = jnp.transpose(coords, (0, 2, 1))
    grid = (_B, _N // _R)
    return pl.pallas_call(
        _knn_body,
        grid=grid,
        in_specs=[
            pl.BlockSpec((1, _R, 3), lambda b, i: (b, i, 0)),
            pl.BlockSpec((1, 3, _N), lambda b, i: (b, 0, 0)),
        ],
        out_specs=[
            pl.BlockSpec((1, _R, _K), lambda b, i: (b, i, 0)),
            pl.BlockSpec((1, _R, _K), lambda b, i: (b, i, 0)),
        ],
        out_shape=[
            jax.ShapeDtypeStruct((_B, _N, _K), jnp.int32),
            jax.ShapeDtypeStruct((_B, _N, _K), jnp.float32),
        ],
    )(coords, coords_t)


_NC = 2
_NS = 16
_NW = _NC * _NS
_PW = _TOTAL // _NW
_CH = 64
_GL = 128
_NG = (_CH * _K) // _GL


def _tv_body(feats_hbm, idx_hbm, w_hbm, tv_hbm, idx_v, rows_v, of_v, w_v,
             tv_v, sem):
    wid = lax.axis_index("s") * _NC + lax.axis_index("c")

    def chunk_body(c, carry):
        base_pt = pl.multiple_of(wid * _PW + c * _CH, _CH)
        base_row = pl.multiple_of(base_pt * _K // _GL, 8)
        pltpu.sync_copy(idx_hbm.at[pl.ds(base_row, _NG)], idx_v)
        pltpu.sync_copy(w_hbm.at[pl.ds(base_pt * _K, _CH * _K)], w_v)
        pltpu.sync_copy(feats_hbm.at[pl.ds(base_pt, _CH)], of_v)
        copies = []
        for g in range(_NG):
            copies.append(
                pltpu.async_copy(
                    feats_hbm.at[idx_v.at[g]],
                    rows_v.at[pl.ds(g * _GL, _GL)],
                    sem,
                ))
        for cp in copies:
            cp.wait()

        lane = lax.broadcasted_iota(jnp.int32, (16,), 0)

        def pt_body(p, vec):
            o0 = of_v[p, pl.ds(0, 16)]
            o1 = of_v[p, pl.ds(16, 16)]
            wk = w_v[pl.ds(p * _K, _K)]
            acc = jnp.zeros((16,), jnp.float32)
            for k in range(_K):
                r = p * _K + k
                d = (jnp.abs(rows_v[r, pl.ds(0, 16)] - o0)
                     + jnp.abs(rows_v[r, pl.ds(16, 16)] - o1))
                acc = acc + d * wk[k]
            for sh in (8, 4, 2, 1):
                acc = acc + acc.at[lane ^ sh].get(mode="promise_in_bounds")
            vec = jnp.where(lane == (p & 15), acc, vec)

            @pl.when((p & 15) == 15)
            def _store():
                tv_v[pl.ds(p - 15, 16)] = vec

            return vec

        lax.fori_loop(0, _CH, pt_body, jnp.zeros((16,), jnp.float32))
        pltpu.sync_copy(tv_v, tv_hbm.at[pl.ds(base_pt, _CH)])
        return carry

    lax.fori_loop(0, _PW // _CH, chunk_body, 0)


@functools.partial(jax.jit, static_argnames=())
def _tv_sc(feats_flat, idx_flat2d, w_flat):
    kern = pl.kernel(
        _tv_body,
        out_type=jax.ShapeDtypeStruct((_TOTAL,), jnp.float32),
        mesh=plsc.VectorSubcoreMesh(core_axis_name="c", subcore_axis_name="s"),
        scratch_types=[
            pltpu.VMEM((_NG, _GL), jnp.int32),
            pltpu.VMEM((_CH * _K, _D), jnp.float32),
            pltpu.VMEM((_CH, _D), jnp.float32),
            pltpu.VMEM((_CH * _K,), jnp.float32),
            pltpu.VMEM((_CH,), jnp.float32),
            pltpu.SemaphoreType.DMA,
        ],
        compiler_params=pltpu.CompilerParams(use_tc_tiling_on_sc=False),
    )
    return kern(feats_flat, idx_flat2d, w_flat)


def kernel(coords, feats, iteration=0):
    coords = lax.stop_gradient(coords)
    nbr_idx, w = _knn(coords)

    feats_flat = feats.reshape(_TOTAL, _D)
    idx_flat2d = nbr_idx.reshape(_TOTAL * _K // _GL, _GL)
    w_flat = w.reshape(_TOTAL * _K)

    tv_flat = _tv_sc(feats_flat, idx_flat2d, w_flat)
    tv = tv_flat.reshape(_B, _N) * _WEIGHT
    total_loss = tv.mean() + jnp.asarray(iteration, jnp.float32) * 0.0
    return (total_loss, tv)
